# baseline (speedup 1.0000x reference)
import jax
import jax.numpy as jnp
from jax.experimental import pallas as pl
from jax.experimental.pallas import tpu as pltpu

_B, _M, _S, _D = 4, 16, 2048, 1024
_S_BLK = 2048


def _body(s_ref, inp_ref, inps_ref, out_ref):
    m = pl.program_id(1)
    s = s_ref[0]
    in_range = jnp.logical_and(m >= s, m < s + _B)

    @pl.when(in_range)
    def _():
        out_ref[...] = inp_ref[...]

    @pl.when(jnp.logical_not(in_range))
    def _():
        out_ref[...] = inps_ref[...]


def _inp_map(c, m, s_ref):
    s = s_ref[0]
    return jnp.clip(m - s, 0, _B - 1), c, 0


def _inps_map(c, m, s_ref):
    s = s_ref[0]
    in_range = jnp.logical_and(m >= s, m < s + _B)
    dead_row = jnp.where(s > 0, s - 1, jnp.minimum(s + _B, _M - 1))
    return jnp.where(in_range, dead_row, m), c, 0


def _out_map(c, m, s_ref):
    return m, c, 0


def kernel(inp, inps, start_idx):
    s = jnp.asarray(start_idx, jnp.int32).reshape((1,))
    grid = (_S // _S_BLK, _M)
    blk = (1, _S_BLK, _D)
    return pl.pallas_call(
        _body,
        grid_spec=pltpu.PrefetchScalarGridSpec(
            num_scalar_prefetch=1,
            grid=grid,
            in_specs=[
                pl.BlockSpec(blk, _inp_map),
                pl.BlockSpec(blk, _inps_map),
            ],
            out_specs=pl.BlockSpec(blk, _out_map),
        ),
        out_shape=jax.ShapeDtypeStruct(inps.shape, inps.dtype),
        compiler_params=pltpu.CompilerParams(vmem_limit_bytes=56 * 1024 * 1024),
    )(s, inp, inps)
